# initial kernel scaffold (unmeasured)
import jax
import jax.numpy as jnp
from jax import lax
from jax.experimental import pallas as pl
from jax.experimental.pallas import tpu as pltpu


def kernel(
    x,
):
    def body(*refs):
        pass

    out_shape = jax.ShapeDtypeStruct(..., jnp.float32)
    return pl.pallas_call(body, out_shape=out_shape)(...)


# baseline (device time: 81973 ns/iter reference)
import jax
import jax.numpy as jnp
from jax import lax
from jax.experimental import pallas as pl
from jax.experimental.pallas import tpu as pltpu

N_Y = 4


def kernel(x):
    m, n = x.shape

    def body(x_ref, out_ref, comm_ref, send_sems, recv_sems):
        my_x = lax.axis_index("x")
        my_y = lax.axis_index("y")
        my_z = lax.axis_index("z")
        left = (my_y - 1) % N_Y
        right = (my_y + 1) % N_Y

        barrier_sem = pltpu.get_barrier_semaphore()
        for nbr in (left, right):
            pl.semaphore_signal(
                barrier_sem,
                inc=1,
                device_id=(my_x, nbr, my_z),
                device_id_type=pl.DeviceIdType.MESH,
            )
        pl.semaphore_wait(barrier_sem, 2)

        comm_ref[0] = x_ref[...].astype(jnp.bfloat16)
        out_ref[...] = x_ref[...]

        for h in range(N_Y - 1):
            rdma = pltpu.make_async_remote_copy(
                src_ref=comm_ref.at[h],
                dst_ref=comm_ref.at[h + 1],
                send_sem=send_sems.at[h],
                recv_sem=recv_sems.at[h],
                device_id=(my_x, right, my_z),
                device_id_type=pl.DeviceIdType.MESH,
            )
            rdma.start()
            rdma.wait()
            out_ref[...] += comm_ref[h + 1].astype(jnp.float32)

    return pl.pallas_call(
        body,
        out_shape=jax.ShapeDtypeStruct((m, n), jnp.float32),
        in_specs=[pl.BlockSpec(memory_space=pltpu.VMEM)],
        out_specs=pl.BlockSpec(memory_space=pltpu.VMEM),
        scratch_shapes=[
            pltpu.VMEM((N_Y, m, n), jnp.bfloat16),
            pltpu.SemaphoreType.DMA((N_Y - 1,)),
            pltpu.SemaphoreType.DMA((N_Y - 1,)),
        ],
        compiler_params=pltpu.CompilerParams(collective_id=0),
    )(x)


# device time: 37796 ns/iter; 2.1688x vs baseline; 2.1688x over previous
import jax
import jax.numpy as jnp
from jax import lax
from jax.experimental import pallas as pl
from jax.experimental.pallas import tpu as pltpu

N_Y = 4
HALF = 1024
K = 4
RPC = HALF // K
_MESH = pl.DeviceIdType.MESH


def kernel(x):
    m, n = x.shape

    def body(
        x_ref,
        out_ref,
        half,
        r1,
        pbuf,
        r2,
        tot,
        zr,
        s1s, r1s, s2s, r2s, s3s, r3s, szs, rzs,
    ):
        my_x = lax.axis_index("x")
        my_y = lax.axis_index("y")
        my_z = lax.axis_index("z")
        zp = lax.rem(my_z, 2)
        zpart = my_z + 1 - 2 * zp
        is_inner = jnp.logical_or(my_y == 1, my_y == 2)
        inner_partner = jnp.where(my_y == 1, 2, 1)
        outer_of_inner = jnp.where(my_y == 1, 0, 3)
        inner_of_outer = jnp.where(my_y == 0, 1, 2)

        barrier = pltpu.get_barrier_semaphore()
        pl.semaphore_signal(
            barrier, inc=1, device_id=(my_x, my_y, zpart), device_id_type=_MESH
        )

        @pl.when(is_inner)
        def _():
            pl.semaphore_signal(
                barrier, inc=1,
                device_id=(my_x, outer_of_inner, my_z), device_id_type=_MESH,
            )
            pl.semaphore_signal(
                barrier, inc=1,
                device_id=(my_x, inner_partner, my_z), device_id_type=_MESH,
            )
            pl.semaphore_wait(barrier, 3)

        @pl.when(jnp.logical_not(is_inner))
        def _():
            pl.semaphore_signal(
                barrier, inc=1,
                device_id=(my_x, inner_of_outer, my_z), device_id_type=_MESH,
            )
            pl.semaphore_wait(barrier, 2)

        base = zp * HALF
        obase = HALF - base
        half[...] = x_ref[pl.ds(base, HALF), :].astype(jnp.bfloat16)

        @pl.when(jnp.logical_not(is_inner))
        def _outer():
            sends1 = []
            for c in range(K):
                sl = pl.ds(c * RPC, RPC)
                d = pltpu.make_async_remote_copy(
                    src_ref=half.at[sl],
                    dst_ref=r1.at[sl],
                    send_sem=s1s.at[c],
                    recv_sem=r1s.at[c],
                    device_id=(my_x, inner_of_outer, my_z),
                    device_id_type=_MESH,
                )
                d.start()
                sends1.append(d)
            zsends = []
            for c in range(K):
                sl = pl.ds(c * RPC, RPC)
                d3 = pltpu.make_async_remote_copy(
                    src_ref=tot.at[sl],
                    dst_ref=tot.at[sl],
                    send_sem=s3s.at[c],
                    recv_sem=r3s.at[c],
                    device_id=(my_x, my_y, my_z),
                    device_id_type=_MESH,
                )
                d3.wait_recv()
                dz = pltpu.make_async_remote_copy(
                    src_ref=tot.at[sl],
                    dst_ref=zr.at[sl],
                    send_sem=szs.at[c],
                    recv_sem=rzs.at[c],
                    device_id=(my_x, my_y, zpart),
                    device_id_type=_MESH,
                )
                dz.start()
                zsends.append(dz)
                out_ref[pl.ds(base + c * RPC, RPC), :] = tot[sl, :].astype(
                    jnp.float32
                )
            for c in range(K):
                zsends[c].wait_recv()
                out_ref[pl.ds(obase + c * RPC, RPC), :] = zr[
                    pl.ds(c * RPC, RPC), :
                ].astype(jnp.float32)
            for d in sends1:
                d.wait_send()
            for d in zsends:
                d.wait_send()

        @pl.when(is_inner)
        def _inner():
            d2s = []
            for c in range(K):
                sl = pl.ds(c * RPC, RPC)
                d1 = pltpu.make_async_remote_copy(
                    src_ref=half.at[sl],
                    dst_ref=r1.at[sl],
                    send_sem=s1s.at[c],
                    recv_sem=r1s.at[c],
                    device_id=(my_x, my_y, my_z),
                    device_id_type=_MESH,
                )
                d1.wait_recv()
                p = half[sl, :].astype(jnp.float32) + r1[sl, :].astype(
                    jnp.float32
                )
                pbuf[sl, :] = p.astype(jnp.bfloat16)
                d2 = pltpu.make_async_remote_copy(
                    src_ref=pbuf.at[sl],
                    dst_ref=r2.at[sl],
                    send_sem=s2s.at[c],
                    recv_sem=r2s.at[c],
                    device_id=(my_x, inner_partner, my_z),
                    device_id_type=_MESH,
                )
                d2.start()
                d2s.append(d2)
            d3s = []
            dzs = []
            for c in range(K):
                sl = pl.ds(c * RPC, RPC)
                d2s[c].wait_recv()
                tot32 = pbuf[sl, :].astype(jnp.float32) + r2[sl, :].astype(
                    jnp.float32
                )
                tot[sl, :] = tot32.astype(jnp.bfloat16)
                d3 = pltpu.make_async_remote_copy(
                    src_ref=tot.at[sl],
                    dst_ref=tot.at[sl],
                    send_sem=s3s.at[c],
                    recv_sem=r3s.at[c],
                    device_id=(my_x, outer_of_inner, my_z),
                    device_id_type=_MESH,
                )
                d3.start()
                d3s.append(d3)
                dz = pltpu.make_async_remote_copy(
                    src_ref=tot.at[sl],
                    dst_ref=zr.at[sl],
                    send_sem=szs.at[c],
                    recv_sem=rzs.at[c],
                    device_id=(my_x, my_y, zpart),
                    device_id_type=_MESH,
                )
                dz.start()
                dzs.append(dz)
                out_ref[pl.ds(base + c * RPC, RPC), :] = tot32
            for c in range(K):
                dzs[c].wait_recv()
                out_ref[pl.ds(obase + c * RPC, RPC), :] = zr[
                    pl.ds(c * RPC, RPC), :
                ].astype(jnp.float32)
            for d in d2s + d3s + dzs:
                d.wait_send()

    return pl.pallas_call(
        body,
        out_shape=jax.ShapeDtypeStruct((m, n), jnp.float32),
        in_specs=[pl.BlockSpec(memory_space=pltpu.VMEM)],
        out_specs=pl.BlockSpec(memory_space=pltpu.VMEM),
        scratch_shapes=[
            pltpu.VMEM((HALF, n), jnp.bfloat16),
            pltpu.VMEM((HALF, n), jnp.bfloat16),
            pltpu.VMEM((HALF, n), jnp.bfloat16),
            pltpu.VMEM((HALF, n), jnp.bfloat16),
            pltpu.VMEM((HALF, n), jnp.bfloat16),
            pltpu.VMEM((HALF, n), jnp.bfloat16),
            pltpu.SemaphoreType.DMA((K,)),
            pltpu.SemaphoreType.DMA((K,)),
            pltpu.SemaphoreType.DMA((K,)),
            pltpu.SemaphoreType.DMA((K,)),
            pltpu.SemaphoreType.DMA((K,)),
            pltpu.SemaphoreType.DMA((K,)),
            pltpu.SemaphoreType.DMA((K,)),
            pltpu.SemaphoreType.DMA((K,)),
        ],
        compiler_params=pltpu.CompilerParams(collective_id=0),
    )(x)


# device time: 35370 ns/iter; 2.3176x vs baseline; 1.0686x over previous
import jax
import jax.numpy as jnp
from jax import lax
from jax.experimental import pallas as pl
from jax.experimental.pallas import tpu as pltpu

N_Y = 4
HALF = 1024
K = 8
RPC = HALF // K
_MESH = pl.DeviceIdType.MESH


def kernel(x):
    m, n = x.shape

    def body(
        x_ref,
        out_ref,
        half,
        r1,
        pbuf,
        r2,
        tot,
        zr,
        s1s, r1s, s2s, r2s, s3s, r3s, szs, rzs,
    ):
        my_x = lax.axis_index("x")
        my_y = lax.axis_index("y")
        my_z = lax.axis_index("z")
        zp = lax.rem(my_z, 2)
        zpart = my_z + 1 - 2 * zp
        is_inner = jnp.logical_or(my_y == 1, my_y == 2)
        inner_partner = jnp.where(my_y == 1, 2, 1)
        outer_of_inner = jnp.where(my_y == 1, 0, 3)
        inner_of_outer = jnp.where(my_y == 0, 1, 2)

        barrier = pltpu.get_barrier_semaphore()
        pl.semaphore_signal(
            barrier, inc=1, device_id=(my_x, my_y, zpart), device_id_type=_MESH
        )

        @pl.when(is_inner)
        def _():
            pl.semaphore_signal(
                barrier, inc=1,
                device_id=(my_x, outer_of_inner, my_z), device_id_type=_MESH,
            )
            pl.semaphore_signal(
                barrier, inc=1,
                device_id=(my_x, inner_partner, my_z), device_id_type=_MESH,
            )
            pl.semaphore_wait(barrier, 3)

        @pl.when(jnp.logical_not(is_inner))
        def _():
            pl.semaphore_signal(
                barrier, inc=1,
                device_id=(my_x, inner_of_outer, my_z), device_id_type=_MESH,
            )
            pl.semaphore_wait(barrier, 2)

        base = zp * HALF
        obase = HALF - base
        half[...] = x_ref[pl.ds(base, HALF), :].astype(jnp.bfloat16)

        @pl.when(jnp.logical_not(is_inner))
        def _outer():
            sends1 = []
            for c in range(K):
                sl = pl.ds(c * RPC, RPC)
                d = pltpu.make_async_remote_copy(
                    src_ref=half.at[sl],
                    dst_ref=r1.at[sl],
                    send_sem=s1s.at[c],
                    recv_sem=r1s.at[c],
                    device_id=(my_x, inner_of_outer, my_z),
                    device_id_type=_MESH,
                )
                d.start()
                sends1.append(d)
            zsends = []
            for c in range(K):
                sl = pl.ds(c * RPC, RPC)
                d3 = pltpu.make_async_remote_copy(
                    src_ref=tot.at[sl],
                    dst_ref=tot.at[sl],
                    send_sem=s3s.at[c],
                    recv_sem=r3s.at[c],
                    device_id=(my_x, my_y, my_z),
                    device_id_type=_MESH,
                )
                d3.wait_recv()
                dz = pltpu.make_async_remote_copy(
                    src_ref=tot.at[sl],
                    dst_ref=zr.at[sl],
                    send_sem=szs.at[c],
                    recv_sem=rzs.at[c],
                    device_id=(my_x, my_y, zpart),
                    device_id_type=_MESH,
                )
                dz.start()
                zsends.append(dz)
                out_ref[pl.ds(base + c * RPC, RPC), :] = tot[sl, :].astype(
                    jnp.float32
                )
            for c in range(K):
                zsends[c].wait_recv()
                out_ref[pl.ds(obase + c * RPC, RPC), :] = zr[
                    pl.ds(c * RPC, RPC), :
                ].astype(jnp.float32)
            for d in sends1:
                d.wait_send()
            for d in zsends:
                d.wait_send()

        @pl.when(is_inner)
        def _inner():
            d2s = []
            for c in range(K):
                sl = pl.ds(c * RPC, RPC)
                d1 = pltpu.make_async_remote_copy(
                    src_ref=half.at[sl],
                    dst_ref=r1.at[sl],
                    send_sem=s1s.at[c],
                    recv_sem=r1s.at[c],
                    device_id=(my_x, my_y, my_z),
                    device_id_type=_MESH,
                )
                d1.wait_recv()
                p = half[sl, :].astype(jnp.float32) + r1[sl, :].astype(
                    jnp.float32
                )
                pbuf[sl, :] = p.astype(jnp.bfloat16)
                d2 = pltpu.make_async_remote_copy(
                    src_ref=pbuf.at[sl],
                    dst_ref=r2.at[sl],
                    send_sem=s2s.at[c],
                    recv_sem=r2s.at[c],
                    device_id=(my_x, inner_partner, my_z),
                    device_id_type=_MESH,
                )
                d2.start()
                d2s.append(d2)
            d3s = []
            dzs = []
            for c in range(K):
                sl = pl.ds(c * RPC, RPC)
                d2s[c].wait_recv()
                tot32 = pbuf[sl, :].astype(jnp.float32) + r2[sl, :].astype(
                    jnp.float32
                )
                tot[sl, :] = tot32.astype(jnp.bfloat16)
                d3 = pltpu.make_async_remote_copy(
                    src_ref=tot.at[sl],
                    dst_ref=tot.at[sl],
                    send_sem=s3s.at[c],
                    recv_sem=r3s.at[c],
                    device_id=(my_x, outer_of_inner, my_z),
                    device_id_type=_MESH,
                )
                d3.start()
                d3s.append(d3)
                dz = pltpu.make_async_remote_copy(
                    src_ref=tot.at[sl],
                    dst_ref=zr.at[sl],
                    send_sem=szs.at[c],
                    recv_sem=rzs.at[c],
                    device_id=(my_x, my_y, zpart),
                    device_id_type=_MESH,
                )
                dz.start()
                dzs.append(dz)
                out_ref[pl.ds(base + c * RPC, RPC), :] = tot32
            for c in range(K):
                dzs[c].wait_recv()
                out_ref[pl.ds(obase + c * RPC, RPC), :] = zr[
                    pl.ds(c * RPC, RPC), :
                ].astype(jnp.float32)
            for d in d2s + d3s + dzs:
                d.wait_send()

    return pl.pallas_call(
        body,
        out_shape=jax.ShapeDtypeStruct((m, n), jnp.float32),
        in_specs=[pl.BlockSpec(memory_space=pltpu.VMEM)],
        out_specs=pl.BlockSpec(memory_space=pltpu.VMEM),
        scratch_shapes=[
            pltpu.VMEM((HALF, n), jnp.bfloat16),
            pltpu.VMEM((HALF, n), jnp.bfloat16),
            pltpu.VMEM((HALF, n), jnp.bfloat16),
            pltpu.VMEM((HALF, n), jnp.bfloat16),
            pltpu.VMEM((HALF, n), jnp.bfloat16),
            pltpu.VMEM((HALF, n), jnp.bfloat16),
            pltpu.SemaphoreType.DMA((K,)),
            pltpu.SemaphoreType.DMA((K,)),
            pltpu.SemaphoreType.DMA((K,)),
            pltpu.SemaphoreType.DMA((K,)),
            pltpu.SemaphoreType.DMA((K,)),
            pltpu.SemaphoreType.DMA((K,)),
            pltpu.SemaphoreType.DMA((K,)),
            pltpu.SemaphoreType.DMA((K,)),
        ],
        compiler_params=pltpu.CompilerParams(collective_id=0),
    )(x)
